# Initial kernel scaffold; baseline (speedup 1.0000x reference)
import jax
import jax.numpy as jnp
from jax.experimental import pallas as pl


def kernel(x, y, word_to_embedding, embedding_to_context, negative_samples):
    raise NotImplementedError("write your pallas kernel here")


# speedup vs baseline: 1.0710x; 1.0710x over previous
import functools

import jax
import jax.numpy as jnp
from jax import lax
from jax.experimental import pallas as pl
from jax.experimental.pallas import tpu as pltpu
from jax.experimental.pallas import tpu_sc as plsc

_VOCAB = 100000
_EMB = 128
_BATCH = 16384
_NEG = 20
_NEG_PAD = 32


def _sc_gather(x_idx, y_idx, neg_idx, w1, w2):
    info = plsc.get_sparse_core_info()
    nc, ns = info.num_cores, info.num_subcores
    nw = nc * ns
    bpw = _BATCH // nw
    ch = 128
    nch = bpw // ch

    @functools.partial(
        pl.kernel,
        out_type=(
            jax.ShapeDtypeStruct((_BATCH, _EMB), jnp.float32),
            jax.ShapeDtypeStruct((_BATCH, _EMB), jnp.float32),
            jax.ShapeDtypeStruct((_NEG_PAD, _EMB), jnp.float32),
        ),
        mesh=plsc.VectorSubcoreMesh(core_axis_name="c", subcore_axis_name="s"),
        scratch_types=[
            pltpu.VMEM((ch,), jnp.int32),
            pltpu.VMEM((ch, _EMB), jnp.float32),
            pltpu.VMEM((_NEG_PAD,), jnp.int32),
            pltpu.VMEM((_NEG_PAD, _EMB), jnp.float32),
            pltpu.SemaphoreType.DMA,
        ],
    )
    def gather_kernel(xi, yi, ni, w1h, w2h, xo, yo, no,
                      idx_v, rows_v, nidx_v, nrows_v, sem):
        wid = lax.axis_index("s") * nc + lax.axis_index("c")
        base = wid * bpw
        for c in range(nch):
            off = base + c * ch
            pltpu.sync_copy(xi.at[pl.ds(off, ch)], idx_v)
            pltpu.async_copy(w1h.at[idx_v], rows_v, sem).wait()
            pltpu.sync_copy(rows_v, xo.at[pl.ds(off, ch)])
            pltpu.sync_copy(yi.at[pl.ds(off, ch)], idx_v)
            pltpu.async_copy(w2h.at[idx_v], rows_v, sem).wait()
            pltpu.sync_copy(rows_v, yo.at[pl.ds(off, ch)])

        @pl.when(wid == 0)
        def _():
            pltpu.sync_copy(ni, nidx_v)
            pltpu.async_copy(w2h.at[nidx_v], nrows_v, sem).wait()
            pltpu.sync_copy(nrows_v, no)

    return gather_kernel(x_idx, y_idx, neg_idx, w1, w2)


def _tc_loss(x_emb, y_emb, neg_emb):
    blk = 2048
    nblk = _BATCH // blk

    def logsig(z):
        return jnp.minimum(z, 0.0) - jnp.log1p(jnp.exp(-jnp.abs(z)))

    def body(neg_ref, x_ref, y_ref, o_ref, acc_ref):
        i = pl.program_id(0)

        @pl.when(i == 0)
        def _():
            acc_ref[0] = 0.0

        x = x_ref[...]
        y = y_ref[...]
        pos = jnp.sum(x * y, axis=1, keepdims=True)
        scores = -lax.dot_general(
            x, neg_ref[...], (((1,), (1,)), ((), ())),
            preferred_element_type=jnp.float32)
        mask = lax.broadcasted_iota(jnp.int32, scores.shape, 1) < _NEG
        tot = jnp.sum(logsig(pos)) + jnp.sum(
            jnp.where(mask, logsig(scores), 0.0))
        acc_ref[0] = acc_ref[0] + tot

        @pl.when(i == nblk - 1)
        def _():
            o_ref[0, 0] = -acc_ref[0] / _BATCH

    out = pl.pallas_call(
        body,
        grid=(nblk,),
        in_specs=[
            pl.BlockSpec((_NEG_PAD, _EMB), lambda i: (0, 0)),
            pl.BlockSpec((blk, _EMB), lambda i: (i, 0)),
            pl.BlockSpec((blk, _EMB), lambda i: (i, 0)),
        ],
        out_specs=pl.BlockSpec(memory_space=pltpu.SMEM),
        out_shape=jax.ShapeDtypeStruct((1, 1), jnp.float32),
        scratch_shapes=[pltpu.SMEM((1,), jnp.float32)],
    )(neg_emb, x_emb, y_emb)
    return out.reshape(())


def kernel(x, y, word_to_embedding, embedding_to_context, negative_samples):
    neg_idx = jnp.zeros((_NEG_PAD,), jnp.int32).at[:_NEG].set(
        negative_samples.astype(jnp.int32))
    x_emb, y_emb, neg_emb = _sc_gather(
        x.astype(jnp.int32), y.astype(jnp.int32), neg_idx,
        word_to_embedding, embedding_to_context)
    return _tc_loss(x_emb, y_emb, neg_emb)


# speedup vs baseline: 1.2792x; 1.1944x over previous
import functools

import jax
import jax.numpy as jnp
from jax import lax
from jax.experimental import pallas as pl
from jax.experimental.pallas import tpu as pltpu
from jax.experimental.pallas import tpu_sc as plsc

_VOCAB = 100000
_EMB = 128
_BATCH = 16384
_NEG = 20
_NEG_PAD = 32


def _sc_gather(x_idx, y_idx, neg_idx, w1, w2):
    info = plsc.get_sparse_core_info()
    nc, ns = info.num_cores, info.num_subcores
    nw = nc * ns
    bpw = _BATCH // nw
    ch = bpw // 2

    @functools.partial(
        pl.kernel,
        out_type=(
            jax.ShapeDtypeStruct((_BATCH, _EMB), jnp.float32),
            jax.ShapeDtypeStruct((_BATCH, _EMB), jnp.float32),
            jax.ShapeDtypeStruct((_NEG_PAD, _EMB), jnp.float32),
        ),
        mesh=plsc.VectorSubcoreMesh(core_axis_name="c", subcore_axis_name="s"),
        scratch_types=[
            pltpu.VMEM((bpw,), jnp.int32),
            pltpu.VMEM((bpw,), jnp.int32),
            pltpu.VMEM((ch, _EMB), jnp.float32),
            pltpu.VMEM((ch, _EMB), jnp.float32),
            pltpu.VMEM((_NEG_PAD,), jnp.int32),
            pltpu.VMEM((_NEG_PAD, _EMB), jnp.float32),
            pltpu.SemaphoreType.DMA,
            pltpu.SemaphoreType.DMA,
            pltpu.SemaphoreType.DMA,
            pltpu.SemaphoreType.DMA,
        ],
    )
    def gather_kernel(xi, yi, ni, w1h, w2h, xo, yo, no,
                      idx_x, idx_y, buf_a, buf_b, nidx_v, nrows_v,
                      gsem, wsem_a, wsem_b, isem):
        wid = lax.axis_index("s") * nc + lax.axis_index("c")
        base = wid * bpw
        ix = pltpu.async_copy(xi.at[pl.ds(base, bpw)], idx_x, isem)
        iy = pltpu.async_copy(yi.at[pl.ds(base, bpw)], idx_y, isem)
        ix.wait()
        g0 = pltpu.async_copy(w1h.at[idx_x.at[pl.ds(0, ch)]], buf_a, gsem)
        g1 = pltpu.async_copy(w1h.at[idx_x.at[pl.ds(ch, ch)]], buf_b, gsem)
        iy.wait()
        g0.wait()
        w0 = pltpu.async_copy(buf_a, xo.at[pl.ds(base, ch)], wsem_a)
        g1.wait()
        w1 = pltpu.async_copy(buf_b, xo.at[pl.ds(base + ch, ch)], wsem_b)
        w0.wait()
        g2 = pltpu.async_copy(w2h.at[idx_y.at[pl.ds(0, ch)]], buf_a, gsem)
        w1.wait()
        g3 = pltpu.async_copy(w2h.at[idx_y.at[pl.ds(ch, ch)]], buf_b, gsem)
        g2.wait()
        w2 = pltpu.async_copy(buf_a, yo.at[pl.ds(base, ch)], wsem_a)
        g3.wait()
        w3 = pltpu.async_copy(buf_b, yo.at[pl.ds(base + ch, ch)], wsem_b)

        @pl.when(wid == 0)
        def _():
            pltpu.sync_copy(ni, nidx_v)
            pltpu.async_copy(w2h.at[nidx_v], nrows_v, gsem).wait()
            pltpu.sync_copy(nrows_v, no)

        w2.wait()
        w3.wait()

    return gather_kernel(x_idx, y_idx, neg_idx, w1, w2)


def _tc_loss(x_emb, y_emb, neg_emb):
    blk = 2048
    nblk = _BATCH // blk

    def logsig(z):
        return jnp.minimum(z, 0.0) - jnp.log1p(jnp.exp(-jnp.abs(z)))

    def body(neg_ref, x_ref, y_ref, o_ref, acc_ref):
        i = pl.program_id(0)

        @pl.when(i == 0)
        def _():
            acc_ref[0] = 0.0

        x = x_ref[...]
        y = y_ref[...]
        pos = jnp.sum(x * y, axis=1, keepdims=True)
        scores = -lax.dot_general(
            x, neg_ref[...], (((1,), (1,)), ((), ())),
            preferred_element_type=jnp.float32)
        mask = lax.broadcasted_iota(jnp.int32, scores.shape, 1) < _NEG
        tot = jnp.sum(logsig(pos)) + jnp.sum(
            jnp.where(mask, logsig(scores), 0.0))
        acc_ref[0] = acc_ref[0] + tot

        @pl.when(i == nblk - 1)
        def _():
            o_ref[0, 0] = -acc_ref[0] / _BATCH

    out = pl.pallas_call(
        body,
        grid=(nblk,),
        in_specs=[
            pl.BlockSpec((_NEG_PAD, _EMB), lambda i: (0, 0)),
            pl.BlockSpec((blk, _EMB), lambda i: (i, 0)),
            pl.BlockSpec((blk, _EMB), lambda i: (i, 0)),
        ],
        out_specs=pl.BlockSpec(memory_space=pltpu.SMEM),
        out_shape=jax.ShapeDtypeStruct((1, 1), jnp.float32),
        scratch_shapes=[pltpu.SMEM((1,), jnp.float32)],
    )(neg_emb, x_emb, y_emb)
    return out.reshape(())


def kernel(x, y, word_to_embedding, embedding_to_context, negative_samples):
    neg_idx = jnp.zeros((_NEG_PAD,), jnp.int32).at[:_NEG].set(
        negative_samples.astype(jnp.int32))
    x_emb, y_emb, neg_emb = _sc_gather(
        x.astype(jnp.int32), y.astype(jnp.int32), neg_idx,
        word_to_embedding, embedding_to_context)
    return _tc_loss(x_emb, y_emb, neg_emb)
